# baseline (speedup 1.0000x reference)
import functools

import jax
import jax.numpy as jnp
from jax import lax
from jax.experimental import pallas as pl
from jax.experimental.pallas import tpu as pltpu
from jax.experimental.pallas import tpu_sc as plsc

N = 10000
NP = 10240
D = 256
H = 128
B = 1024
OBS = 512
E = 320000
EP = 327680
NS = 16
K = 128
NB = 16
R = 2
CHUNKS_PER_SC = EP // K // 2
CHUNKS_PER_SUB = CHUNKS_PER_SC // NS
ROWS_PER_SUB = NP // NS


def _encode_body(x_ref, w_ref, b_ref, e_ref):
    y = jnp.dot(x_ref[...], w_ref[...], preferred_element_type=jnp.float32)
    y = jnp.tanh(y + b_ref[...]).astype(jnp.bfloat16)
    e_ref[:, 0, :] = y[:, :H]
    e_ref[:, 1, :] = y[:, H:]


def _encode(node_feat_p, w_node, b_node2):
    blk = NP // 4
    return pl.pallas_call(
        _encode_body,
        grid=(4,),
        in_specs=[
            pl.BlockSpec((blk, H), lambda i: (i, 0)),
            pl.BlockSpec((H, D), lambda i: (0, 0)),
            pl.BlockSpec((1, D), lambda i: (0, 0)),
        ],
        out_specs=[pl.BlockSpec((blk, 2, H), lambda i: (i, 0, 0))],
        out_shape=[jax.ShapeDtypeStruct((NP, 2, H), jnp.bfloat16)],
    )(node_feat_p, w_node, b_node2)[0]


def _obs_body(obs_ref, wo_ref, bo_ref, of_ref):
    of_ref[...] = jnp.tanh(
        jnp.dot(obs_ref[...], wo_ref[...], preferred_element_type=jnp.float32)
        + bo_ref[...]
    )


def _obs_proj(obs_vec, w_obs, b_obs2):
    return pl.pallas_call(
        _obs_body,
        out_shape=[jax.ShapeDtypeStruct((B, D), jnp.float32)],
    )(obs_vec, w_obs, b_obs2)[0]


def _sc_message(enc3, zro3, src2, dst2):
    mesh = plsc.VectorSubcoreMesh(core_axis_name="c", subcore_axis_name="s")

    @functools.partial(
        pl.kernel,
        mesh=mesh,
        compiler_params=pltpu.CompilerParams(use_tc_tiling_on_sc=False),
        out_type=[jax.ShapeDtypeStruct((NP, 2, H), jnp.bfloat16)] * 2,
        scratch_types=[
            pltpu.VMEM((NB, K), jnp.int32),
            pltpu.VMEM((NB, K), jnp.int32),
            pltpu.VMEM((R, K, 2, H), jnp.bfloat16),
            pltpu.VMEM_SHARED((NP, 2, H), jnp.bfloat16),
            pltpu.SemaphoreType.DMA,
            pltpu.SemaphoreType.DMA,
        ],
    )
    def k(enc_hbm, zro_hbm, src_hbm, dst_hbm, p0_hbm, p1_hbm,
          sidx, didx, rows, acc, sem0, sem1):
        c = lax.axis_index("c")
        s = lax.axis_index("s")
        row0 = s * ROWS_PER_SUB
        sems_g = (sem0, sem1)

        @pl.when(c == 0)
        def _():
            pltpu.sync_copy(
                enc_hbm.at[pl.ds(row0, ROWS_PER_SUB)],
                acc.at[pl.ds(row0, ROWS_PER_SUB)],
            )

        @pl.when(c == 1)
        def _():
            pltpu.sync_copy(
                zro_hbm.at[pl.ds(row0, ROWS_PER_SUB)],
                acc.at[pl.ds(row0, ROWS_PER_SUB)],
            )

        plsc.subcore_barrier()

        def block(b, carry):
            base_row = c * CHUNKS_PER_SC + s * CHUNKS_PER_SUB + b * NB
            pltpu.sync_copy(src_hbm.at[pl.ds(base_row, NB)], sidx)
            pltpu.sync_copy(dst_hbm.at[pl.ds(base_row, NB)], didx)
            hg = [None] * NB
            for j in range(R):
                hg[j] = pltpu.async_copy(
                    enc_hbm.at[sidx.at[j]], rows.at[j], sems_g[j])
            for j in range(NB):
                hg[j].wait()
                pltpu.sync_copy(rows.at[j % R], acc.at[didx.at[j]],
                                add=True)
                nj = j + R
                if nj < NB:
                    hg[nj] = pltpu.async_copy(
                        enc_hbm.at[sidx.at[nj]], rows.at[nj % R],
                        sems_g[nj % R])
            return carry

        lax.fori_loop(0, CHUNKS_PER_SUB // NB, block, 0)

        plsc.subcore_barrier()

        @pl.when(c == 0)
        def _():
            pltpu.sync_copy(
                acc.at[pl.ds(row0, ROWS_PER_SUB)],
                p0_hbm.at[pl.ds(row0, ROWS_PER_SUB)],
            )

        @pl.when(c == 1)
        def _():
            pltpu.sync_copy(
                acc.at[pl.ds(row0, ROWS_PER_SUB)],
                p1_hbm.at[pl.ds(row0, ROWS_PER_SUB)],
            )

    return k(enc3, zro3, src2, dst2)


def _pool_body(p0_ref, p1_ref, wg_ref, bg_ref, attn_ref, pooled_ref):
    nr0 = (p0_ref[:N, :H].astype(jnp.float32)
           + p1_ref[:N, :H].astype(jnp.float32))
    nr1 = (p0_ref[:N, H:].astype(jnp.float32)
           + p1_ref[:N, H:].astype(jnp.float32))
    sc = (
        jnp.dot(jnp.tanh(nr0), wg_ref[:H, :],
                preferred_element_type=jnp.float32)
        + jnp.dot(jnp.tanh(nr1), wg_ref[H:, :],
                  preferred_element_type=jnp.float32)
        + bg_ref[...]
    )
    m = jnp.max(sc)
    ex = jnp.exp(sc - m)
    attn = ex / jnp.sum(ex)
    attn_ref[...] = attn[:, 0]
    pool0 = lax.dot_general(attn, nr0, (((0,), (0,)), ((), ())),
                            preferred_element_type=jnp.float32)
    pool1 = lax.dot_general(attn, nr1, (((0,), (0,)), ((), ())),
                            preferred_element_type=jnp.float32)
    pooled_ref[...] = jnp.concatenate([pool0, pool1], axis=1)


def _pool(p0, p1, w_gate, b_gate2):
    return pl.pallas_call(
        _pool_body,
        out_shape=[
            jax.ShapeDtypeStruct((N,), jnp.float32),
            jax.ShapeDtypeStruct((1, D), jnp.float32),
        ],
    )(p0, p1, w_gate, b_gate2)


def _state_body(of_ref, pooled_ref, wc_ref, bc_ref, state_ref):
    g = jnp.dot(pooled_ref[...], wc_ref[D:, :],
                preferred_element_type=jnp.float32)
    state_ref[...] = jnp.tanh(
        jnp.dot(of_ref[...], wc_ref[:D, :], preferred_element_type=jnp.float32)
        + g + bc_ref[...]
    )


def _state(obs_feat, pooled, w_comb, b_comb2):
    return pl.pallas_call(
        _state_body,
        out_shape=[jax.ShapeDtypeStruct((B, D), jnp.float32)],
    )(obs_feat, pooled, w_comb, b_comb2)[0]


def kernel(obs_vec, node_feat, edge_index, W_obs, b_obs, W_node, b_node,
           W_gate, b_gate, W_comb, b_comb):
    node_feat_p = jnp.pad(node_feat, ((0, NP - N), (0, 0)))
    pad = EP - E
    pad_iota = jnp.arange(pad, dtype=jnp.int32)
    src2 = jnp.concatenate([edge_index[:, 0], pad_iota % N]).reshape(
        EP // K, K)
    dst2 = jnp.concatenate([edge_index[:, 1], N + pad_iota % (NP - N)]
                           ).reshape(EP // K, K)

    enc3 = _encode(node_feat_p, W_node, b_node.reshape(1, D))
    obs_feat = _obs_proj(obs_vec, W_obs, b_obs.reshape(1, D))
    zro3 = jnp.zeros((NP, 2, H), jnp.bfloat16)
    p0, p1 = _sc_message(enc3, zro3, src2, dst2)
    attn, pooled = _pool(p0.reshape(NP, D), p1.reshape(NP, D),
                         W_gate, b_gate.reshape(1, 1))
    state = _state(obs_feat, pooled, W_comb, b_comb.reshape(1, D))
    return state, attn


# speedup vs baseline: 1.0626x; 1.0626x over previous
import functools

import jax
import jax.numpy as jnp
from jax import lax
from jax.experimental import pallas as pl
from jax.experimental.pallas import tpu as pltpu
from jax.experimental.pallas import tpu_sc as plsc

N = 10000
NP = 10240
D = 256
H = 128
B = 1024
OBS = 512
E = 320000
EP = 327680
NS = 16
K = 128
NB = 16
R = 2
CHUNKS_PER_SC = EP // K // 2
CHUNKS_PER_SUB = CHUNKS_PER_SC // NS
ROWS_PER_SUB = NP // NS


def _encode_body(x_ref, w_ref, b_ref, e_ref):
    y = jnp.dot(x_ref[...], w_ref[...], preferred_element_type=jnp.float32)
    e_ref[...] = jnp.tanh(y + b_ref[...]).astype(jnp.bfloat16)


def _encode(node_feat_p, w_node, b_node2):
    blk = NP // 4
    return pl.pallas_call(
        _encode_body,
        grid=(4,),
        in_specs=[
            pl.BlockSpec((blk, H), lambda i: (i, 0)),
            pl.BlockSpec((H, D), lambda i: (0, 0)),
            pl.BlockSpec((1, D), lambda i: (0, 0)),
        ],
        out_specs=[pl.BlockSpec((blk, D), lambda i: (i, 0))],
        out_shape=[jax.ShapeDtypeStruct((NP, D), jnp.bfloat16)],
    )(node_feat_p, w_node, b_node2)[0]


def _obs_body(obs_ref, wo_ref, bo_ref, of_ref):
    of_ref[...] = jnp.tanh(
        jnp.dot(obs_ref[...], wo_ref[...], preferred_element_type=jnp.float32)
        + bo_ref[...]
    )


def _obs_proj(obs_vec, w_obs, b_obs2):
    return pl.pallas_call(
        _obs_body,
        out_shape=[jax.ShapeDtypeStruct((B, D), jnp.float32)],
    )(obs_vec, w_obs, b_obs2)[0]


def _sc_message(enc3, zro3, src2, dst2):
    mesh = plsc.VectorSubcoreMesh(core_axis_name="c", subcore_axis_name="s")

    @functools.partial(
        pl.kernel,
        mesh=mesh,
        compiler_params=pltpu.CompilerParams(use_tc_tiling_on_sc=False),
        out_type=[jax.ShapeDtypeStruct((NP, D), jnp.bfloat16)] * 2,
        scratch_types=[
            pltpu.VMEM((NB, K), jnp.int32),
            pltpu.VMEM((NB, K), jnp.int32),
            pltpu.VMEM((R, K, D), jnp.bfloat16),
            pltpu.VMEM_SHARED((NP, D), jnp.bfloat16),
            pltpu.SemaphoreType.DMA,
            pltpu.SemaphoreType.DMA,
        ],
    )
    def k(enc_hbm, zro_hbm, src_hbm, dst_hbm, p0_hbm, p1_hbm,
          sidx, didx, rows, acc, sem0, sem1):
        c = lax.axis_index("c")
        s = lax.axis_index("s")
        row0 = s * ROWS_PER_SUB
        sems_g = (sem0, sem1)

        @pl.when(c == 0)
        def _():
            pltpu.sync_copy(
                enc_hbm.at[pl.ds(row0, ROWS_PER_SUB)],
                acc.at[pl.ds(row0, ROWS_PER_SUB)],
            )

        @pl.when(c == 1)
        def _():
            pltpu.sync_copy(
                zro_hbm.at[pl.ds(row0, ROWS_PER_SUB)],
                acc.at[pl.ds(row0, ROWS_PER_SUB)],
            )

        plsc.subcore_barrier()

        def block(b, carry):
            base_row = c * CHUNKS_PER_SC + s * CHUNKS_PER_SUB + b * NB
            pltpu.sync_copy(src_hbm.at[pl.ds(base_row, NB)], sidx)
            pltpu.sync_copy(dst_hbm.at[pl.ds(base_row, NB)], didx)
            hg = [None] * NB
            for j in range(R):
                hg[j] = pltpu.async_copy(
                    enc_hbm.at[sidx.at[j]], rows.at[j], sems_g[j])
            for j in range(NB):
                hg[j].wait()
                pltpu.sync_copy(rows.at[j % R], acc.at[didx.at[j]],
                                add=True)
                nj = j + R
                if nj < NB:
                    hg[nj] = pltpu.async_copy(
                        enc_hbm.at[sidx.at[nj]], rows.at[nj % R],
                        sems_g[nj % R])
            return carry

        lax.fori_loop(0, CHUNKS_PER_SUB // NB, block, 0)

        plsc.subcore_barrier()

        @pl.when(c == 0)
        def _():
            pltpu.sync_copy(
                acc.at[pl.ds(row0, ROWS_PER_SUB)],
                p0_hbm.at[pl.ds(row0, ROWS_PER_SUB)],
            )

        @pl.when(c == 1)
        def _():
            pltpu.sync_copy(
                acc.at[pl.ds(row0, ROWS_PER_SUB)],
                p1_hbm.at[pl.ds(row0, ROWS_PER_SUB)],
            )

    return k(enc3, zro3, src2, dst2)


def _pool_body(p0_ref, p1_ref, wg_ref, bg_ref, attn_ref, pooled_ref):
    nr0 = (p0_ref[:N, :H].astype(jnp.float32)
           + p1_ref[:N, :H].astype(jnp.float32))
    nr1 = (p0_ref[:N, H:].astype(jnp.float32)
           + p1_ref[:N, H:].astype(jnp.float32))
    sc = (
        jnp.dot(jnp.tanh(nr0), wg_ref[:H, :],
                preferred_element_type=jnp.float32)
        + jnp.dot(jnp.tanh(nr1), wg_ref[H:, :],
                  preferred_element_type=jnp.float32)
        + bg_ref[...]
    )
    m = jnp.max(sc)
    ex = jnp.exp(sc - m)
    attn = ex / jnp.sum(ex)
    attn_ref[...] = attn[:, 0]
    pool0 = lax.dot_general(attn, nr0, (((0,), (0,)), ((), ())),
                            preferred_element_type=jnp.float32)
    pool1 = lax.dot_general(attn, nr1, (((0,), (0,)), ((), ())),
                            preferred_element_type=jnp.float32)
    pooled_ref[...] = jnp.concatenate([pool0, pool1], axis=1)


def _pool(p0, p1, w_gate, b_gate2):
    return pl.pallas_call(
        _pool_body,
        out_shape=[
            jax.ShapeDtypeStruct((N,), jnp.float32),
            jax.ShapeDtypeStruct((1, D), jnp.float32),
        ],
    )(p0, p1, w_gate, b_gate2)


def _state_body(of_ref, pooled_ref, wc_ref, bc_ref, state_ref):
    g = jnp.dot(pooled_ref[...], wc_ref[D:, :],
                preferred_element_type=jnp.float32)
    state_ref[...] = jnp.tanh(
        jnp.dot(of_ref[...], wc_ref[:D, :], preferred_element_type=jnp.float32)
        + g + bc_ref[...]
    )


def _state(obs_feat, pooled, w_comb, b_comb2):
    return pl.pallas_call(
        _state_body,
        out_shape=[jax.ShapeDtypeStruct((B, D), jnp.float32)],
    )(obs_feat, pooled, w_comb, b_comb2)[0]


def kernel(obs_vec, node_feat, edge_index, W_obs, b_obs, W_node, b_node,
           W_gate, b_gate, W_comb, b_comb):
    node_feat_p = jnp.pad(node_feat, ((0, NP - N), (0, 0)))
    pad = EP - E
    pad_iota = jnp.arange(pad, dtype=jnp.int32)
    src2 = jnp.concatenate([edge_index[:, 0], pad_iota % N]).reshape(
        EP // K, K)
    dst2 = jnp.concatenate([edge_index[:, 1], N + pad_iota % (NP - N)]
                           ).reshape(EP // K, K)

    enc3 = _encode(node_feat_p, W_node, b_node.reshape(1, D))
    obs_feat = _obs_proj(obs_vec, W_obs, b_obs.reshape(1, D))
    zro3 = jnp.zeros((NP, D), jnp.bfloat16)
    p0, p1 = _sc_message(enc3, zro3, src2, dst2)
    attn, pooled = _pool(p0, p1, W_gate, b_gate.reshape(1, 1))
    state = _state(obs_feat, pooled, W_comb, b_comb.reshape(1, D))
    return state, attn
